# baseline (speedup 1.0000x reference)
import jax
import jax.numpy as jnp
from jax.experimental import pallas as pl
from jax.experimental.pallas import tpu as pltpu

N = 4096
M = 4096
NT = 512
MC = 256
ICP_ITERS = 10
NSWEEP = 6
BIGF = 1e30

F32 = jnp.float32


def _align(sx, sy, sz, tx, ty, tz):
    invn = F32(1.0 / N)
    smx = jnp.sum(sx, axis=1, keepdims=True) * invn
    smy = jnp.sum(sy, axis=1, keepdims=True) * invn
    smz = jnp.sum(sz, axis=1, keepdims=True) * invn
    tmx = jnp.sum(tx, axis=1, keepdims=True) * invn
    tmy = jnp.sum(ty, axis=1, keepdims=True) * invn
    tmz = jnp.sum(tz, axis=1, keepdims=True) * invn
    cxs = sx - smx
    cys = sy - smy
    czs = sz - smz
    cxt = tx - tmx
    cyt = ty - tmy
    czt = tz - tmz

    si = jax.lax.broadcasted_iota(jnp.int32, (3, sx.shape[1]), 0)
    Sc = jnp.where(si == 0, cxs, jnp.where(si == 1, cys, czs))
    Tc = jnp.where(si == 0, cxt, jnp.where(si == 1, cyt, czt))
    H = jax.lax.dot_general(Sc, Tc, (((1,), (1,)), ((), ())),
                            preferred_element_type=F32)
    hxx = H[0:1, 0:1]; hxy = H[0:1, 1:2]; hxz = H[0:1, 2:3]
    hyx = H[1:2, 0:1]; hyy = H[1:2, 1:2]; hyz = H[1:2, 2:3]
    hzx = H[2:3, 0:1]; hzy = H[2:3, 1:2]; hzz = H[2:3, 2:3]

    ri = jax.lax.broadcasted_iota(jnp.int32, (4, 4), 0)
    ci = jax.lax.broadcasted_iota(jnp.int32, (4, 4), 1)
    A = jnp.zeros((4, 4), F32)

    def put(A, i, j, val):
        return A + jnp.where((ri == i) & (ci == j), val, F32(0.0))

    d0 = hxx + hyy + hzz
    d1 = hxx - hyy - hzz
    d2_ = -hxx + hyy - hzz
    d3 = -hxx - hyy + hzz
    o01 = hyz - hzy
    o02 = hzx - hxz
    o03 = hxy - hyx
    o12 = hxy + hyx
    o13 = hzx + hxz
    o23 = hyz + hzy
    A = put(A, 0, 0, d0); A = put(A, 1, 1, d1)
    A = put(A, 2, 2, d2_); A = put(A, 3, 3, d3)
    A = put(A, 0, 1, o01); A = put(A, 1, 0, o01)
    A = put(A, 0, 2, o02); A = put(A, 2, 0, o02)
    A = put(A, 0, 3, o03); A = put(A, 3, 0, o03)
    A = put(A, 1, 2, o12); A = put(A, 2, 1, o12)
    A = put(A, 1, 3, o13); A = put(A, 3, 1, o13)
    A = put(A, 2, 3, o23); A = put(A, 3, 2, o23)

    V0 = jnp.where(ri == ci, F32(1.0), F32(0.0))

    def sweep(_, AV):
        A, V = AV
        for (p, q) in ((0, 1), (0, 2), (0, 3), (1, 2), (1, 3), (2, 3)):
            app = A[p:p + 1, p:p + 1]
            aqq = A[q:q + 1, q:q + 1]
            apq = A[p:p + 1, q:q + 1]
            safe = jnp.where(apq == 0.0, F32(1.0), apq)
            tau = (aqq - app) / (2.0 * safe)
            tt = jnp.where(tau < 0, F32(-1.0), F32(1.0)) / (
                jnp.abs(tau) + jnp.sqrt(1.0 + tau * tau))
            tt = jnp.where(tau == 0.0, F32(1.0), tt)
            tt = jnp.where(apq == 0.0, F32(0.0), tt)
            c = 1.0 / jnp.sqrt(1.0 + tt * tt)
            s = tt * c
            rp = c * A[p:p + 1, :] - s * A[q:q + 1, :]
            rq = s * A[p:p + 1, :] + c * A[q:q + 1, :]
            A = jnp.where(ri == p, rp, A)
            A = jnp.where(ri == q, rq, A)
            cp = c * A[:, p:p + 1] - s * A[:, q:q + 1]
            cq = s * A[:, p:p + 1] + c * A[:, q:q + 1]
            A = jnp.where(ci == p, cp, A)
            A = jnp.where(ci == q, cq, A)
            vp = c * V[:, p:p + 1] - s * V[:, q:q + 1]
            vq = s * V[:, p:p + 1] + c * V[:, q:q + 1]
            V = jnp.where(ci == p, vp, V)
            V = jnp.where(ci == q, vq, V)
        return A, V

    A, V = jax.lax.fori_loop(0, NSWEEP, sweep, (A, V0))

    diag = jnp.sum(jnp.where(ri == ci, A, F32(0.0)), axis=0, keepdims=True)
    dmax = jnp.max(diag, axis=1, keepdims=True)
    ci_row = jax.lax.broadcasted_iota(jnp.int32, (1, 4), 1).astype(F32)
    bsel = jnp.min(jnp.where(diag == dmax, ci_row, F32(9.0)),
                   axis=1, keepdims=True)
    onehot = ci_row == bsel
    q = jnp.sum(jnp.where(onehot, V, F32(0.0)), axis=1, keepdims=True)
    q = q / jnp.sqrt(jnp.sum(q * q, axis=0, keepdims=True))

    qw = q[0:1, 0:1]; qx = q[1:2, 0:1]; qy = q[2:3, 0:1]; qz = q[3:4, 0:1]
    r00 = 1.0 - 2.0 * (qy * qy + qz * qz)
    r01 = 2.0 * (qx * qy - qw * qz)
    r02 = 2.0 * (qx * qz + qw * qy)
    r10 = 2.0 * (qx * qy + qw * qz)
    r11 = 1.0 - 2.0 * (qx * qx + qz * qz)
    r12 = 2.0 * (qy * qz - qw * qx)
    r20 = 2.0 * (qx * qz - qw * qy)
    r21 = 2.0 * (qy * qz + qw * qx)
    r22 = 1.0 - 2.0 * (qx * qx + qy * qy)
    t0 = tmx - (r00 * smx + r01 * smy + r02 * smz)
    t1 = tmy - (r10 * smx + r11 * smy + r12 * smz)
    t2 = tmz - (r20 * smx + r21 * smy + r22 * smz)
    R = (r00, r01, r02, r10, r11, r12, r20, r21, r22)
    return R, (t0, t1, t2), q


def _icp_body(src_ref, tgt_ref, out_ref, tmp_ref, g_ref):
    tmp_ref[...] = src_ref[0]
    si3 = jax.lax.broadcasted_iota(jnp.int32, (3, NT), 0)

    def icp_iter(_, carry):
        def n_body(nt, c2):
            noff = nt * NT
            sx = tmp_ref[0:1, pl.ds(noff, NT)]
            sy = tmp_ref[1:2, pl.ds(noff, NT)]
            sz = tmp_ref[2:3, pl.ds(noff, NT)]
            stile = tmp_ref[:, pl.ds(noff, NT)]
            s2 = sx * sx + sy * sy + sz * sz

            def m_body(c, bd_bi):
                bd, bi = bd_bi
                moff = c * MC
                tx = tgt_ref[0, pl.ds(moff, MC), 0:1]
                ty = tgt_ref[0, pl.ds(moff, MC), 1:2]
                tz = tgt_ref[0, pl.ds(moff, MC), 2:3]
                ttile = tgt_ref[0, pl.ds(moff, MC), :]
                t2 = tx * tx + ty * ty + tz * tz
                cross = jax.lax.dot_general(
                    ttile, stile, (((1,), (0,)), ((), ())),
                    preferred_element_type=F32)
                d2 = jnp.maximum((s2 + t2) - 2.0 * cross, 0.0)
                dmin = jnp.min(d2, axis=0, keepdims=True)
                mi = (jax.lax.broadcasted_iota(jnp.int32, (MC, 1), 0)
                      .astype(F32) + moff.astype(F32))
                idxm = jnp.min(jnp.where(d2 == dmin, mi, F32(BIGF)),
                               axis=0, keepdims=True)
                upd = dmin < bd
                return (jnp.where(upd, dmin, bd), jnp.where(upd, idxm, bi))

            bd0 = jnp.full((1, NT), F32(BIGF))
            bi0 = jnp.zeros((1, NT), F32)
            _, bi = jax.lax.fori_loop(0, M // MC, m_body, (bd0, bi0))

            def g_body(c, g):
                gx, gy, gz = g
                moff = c * MC
                tx = tgt_ref[0, pl.ds(moff, MC), 0:1]
                ty = tgt_ref[0, pl.ds(moff, MC), 1:2]
                tz = tgt_ref[0, pl.ds(moff, MC), 2:3]
                mi = (jax.lax.broadcasted_iota(jnp.int32, (MC, 1), 0)
                      .astype(F32) + moff.astype(F32))
                oh = jnp.where(mi == bi, F32(1.0), F32(0.0))
                gx = gx + jnp.sum(oh * tx, axis=0, keepdims=True)
                gy = gy + jnp.sum(oh * ty, axis=0, keepdims=True)
                gz = gz + jnp.sum(oh * tz, axis=0, keepdims=True)
                return gx, gy, gz

            z = jnp.zeros((1, NT), F32)
            gx, gy, gz = jax.lax.fori_loop(0, M // MC, g_body, (z, z, z))
            gtile = jnp.where(si3 == 0, gx, jnp.where(si3 == 1, gy, gz))
            g_ref[:, pl.ds(noff, NT)] = gtile
            return c2

        jax.lax.fori_loop(0, N // NT, n_body, 0)

        R, t, _ = _align(
            tmp_ref[0:1, :], tmp_ref[1:2, :], tmp_ref[2:3, :],
            g_ref[0:1, :], g_ref[1:2, :], g_ref[2:3, :])
        (r00, r01, r02, r10, r11, r12, r20, r21, r22) = R
        (t0, t1, t2) = t
        ri3 = jax.lax.broadcasted_iota(jnp.int32, (3, 3), 0)
        ci3 = jax.lax.broadcasted_iota(jnp.int32, (3, 3), 1)
        R33 = jnp.zeros((3, 3), F32)
        for (i, j, v) in ((0, 0, r00), (0, 1, r01), (0, 2, r02),
                          (1, 0, r10), (1, 1, r11), (1, 2, r12),
                          (2, 0, r20), (2, 1, r21), (2, 2, r22)):
            R33 = R33 + jnp.where((ri3 == i) & (ci3 == j), v, F32(0.0))
        newP = jax.lax.dot_general(R33, tmp_ref[...], (((1,), (0,)), ((), ())),
                                   preferred_element_type=F32)
        si = jax.lax.broadcasted_iota(jnp.int32, (3, N), 0)
        tb = jnp.where(si == 0, t0, jnp.where(si == 1, t1, t2))
        tmp_ref[...] = newP + tb
        return carry

    jax.lax.fori_loop(0, ICP_ITERS, icp_iter, 0)

    _, t, q = _align(
        src_ref[0, 0:1, :], src_ref[0, 1:2, :], src_ref[0, 2:3, :],
        tmp_ref[0:1, :], tmp_ref[1:2, :], tmp_ref[2:3, :])
    (t0, t1, t2) = t
    absq = jnp.abs(q)
    qmax = jnp.max(absq, axis=0, keepdims=True)
    ri4 = jax.lax.broadcasted_iota(jnp.int32, (4, 1), 0).astype(F32)
    bsel = jnp.min(jnp.where(absq == qmax, ri4, F32(9.0)),
                   axis=0, keepdims=True)
    val = jnp.sum(jnp.where(ri4 == bsel, q, F32(0.0)), axis=0, keepdims=True)
    q = q * jnp.where(val < 0, F32(-1.0), F32(1.0))
    qw = q[0:1, 0:1]; qx = q[1:2, 0:1]; qy = q[2:3, 0:1]; qz = q[3:4, 0:1]

    li = jax.lax.broadcasted_iota(jnp.int32, (1, 8), 1)
    out = jnp.where(li == 0, t0,
          jnp.where(li == 1, t1,
          jnp.where(li == 2, t2,
          jnp.where(li == 3, qx,
          jnp.where(li == 4, qy,
          jnp.where(li == 5, qz,
          jnp.where(li == 6, qw, F32(0.0))))))))
    out_ref[...] = out.reshape(1, 1, 8)


def kernel(source, target):
    B = source.shape[0]
    src_p = jnp.swapaxes(source, 1, 2)
    out3 = pl.pallas_call(
        _icp_body,
        grid=(B,),
        in_specs=[
            pl.BlockSpec((1, 3, N), lambda b: (b, 0, 0)),
            pl.BlockSpec((1, M, 3), lambda b: (b, 0, 0)),
        ],
        out_specs=pl.BlockSpec((1, 1, 8), lambda b: (b, 0, 0)),
        out_shape=jax.ShapeDtypeStruct((B, 1, 8), F32),
        scratch_shapes=[
            pltpu.VMEM((3, N), F32),
            pltpu.VMEM((3, N), F32),
        ],
    )(src_p, target)
    return out3[:, 0, :7]


# speedup vs baseline: 1.0974x; 1.0974x over previous
import jax
import jax.numpy as jnp
from jax.experimental import pallas as pl
from jax.experimental.pallas import tpu as pltpu

N = 4096
M = 4096
NT = 1024
MC = 512
ICP_ITERS = 10
NSWEEP = 6
BIGF = 1e30

F32 = jnp.float32


def _align(sx, sy, sz, tx, ty, tz):
    invn = F32(1.0 / N)
    smx = jnp.sum(sx, axis=1, keepdims=True) * invn
    smy = jnp.sum(sy, axis=1, keepdims=True) * invn
    smz = jnp.sum(sz, axis=1, keepdims=True) * invn
    tmx = jnp.sum(tx, axis=1, keepdims=True) * invn
    tmy = jnp.sum(ty, axis=1, keepdims=True) * invn
    tmz = jnp.sum(tz, axis=1, keepdims=True) * invn
    cxs = sx - smx
    cys = sy - smy
    czs = sz - smz
    cxt = tx - tmx
    cyt = ty - tmy
    czt = tz - tmz

    si = jax.lax.broadcasted_iota(jnp.int32, (3, sx.shape[1]), 0)
    Sc = jnp.where(si == 0, cxs, jnp.where(si == 1, cys, czs))
    Tc = jnp.where(si == 0, cxt, jnp.where(si == 1, cyt, czt))
    H = jax.lax.dot_general(Sc, Tc, (((1,), (1,)), ((), ())),
                            preferred_element_type=F32)
    hxx = H[0:1, 0:1]; hxy = H[0:1, 1:2]; hxz = H[0:1, 2:3]
    hyx = H[1:2, 0:1]; hyy = H[1:2, 1:2]; hyz = H[1:2, 2:3]
    hzx = H[2:3, 0:1]; hzy = H[2:3, 1:2]; hzz = H[2:3, 2:3]

    ri = jax.lax.broadcasted_iota(jnp.int32, (4, 4), 0)
    ci = jax.lax.broadcasted_iota(jnp.int32, (4, 4), 1)
    A = jnp.zeros((4, 4), F32)

    def put(A, i, j, val):
        return A + jnp.where((ri == i) & (ci == j), val, F32(0.0))

    d0 = hxx + hyy + hzz
    d1 = hxx - hyy - hzz
    d2_ = -hxx + hyy - hzz
    d3 = -hxx - hyy + hzz
    o01 = hyz - hzy
    o02 = hzx - hxz
    o03 = hxy - hyx
    o12 = hxy + hyx
    o13 = hzx + hxz
    o23 = hyz + hzy
    A = put(A, 0, 0, d0); A = put(A, 1, 1, d1)
    A = put(A, 2, 2, d2_); A = put(A, 3, 3, d3)
    A = put(A, 0, 1, o01); A = put(A, 1, 0, o01)
    A = put(A, 0, 2, o02); A = put(A, 2, 0, o02)
    A = put(A, 0, 3, o03); A = put(A, 3, 0, o03)
    A = put(A, 1, 2, o12); A = put(A, 2, 1, o12)
    A = put(A, 1, 3, o13); A = put(A, 3, 1, o13)
    A = put(A, 2, 3, o23); A = put(A, 3, 2, o23)

    V0 = jnp.where(ri == ci, F32(1.0), F32(0.0))

    def sweep(_, AV):
        A, V = AV
        for (p, q) in ((0, 1), (0, 2), (0, 3), (1, 2), (1, 3), (2, 3)):
            app = A[p:p + 1, p:p + 1]
            aqq = A[q:q + 1, q:q + 1]
            apq = A[p:p + 1, q:q + 1]
            safe = jnp.where(apq == 0.0, F32(1.0), apq)
            tau = (aqq - app) / (2.0 * safe)
            tt = jnp.where(tau < 0, F32(-1.0), F32(1.0)) / (
                jnp.abs(tau) + jnp.sqrt(1.0 + tau * tau))
            tt = jnp.where(tau == 0.0, F32(1.0), tt)
            tt = jnp.where(apq == 0.0, F32(0.0), tt)
            c = 1.0 / jnp.sqrt(1.0 + tt * tt)
            s = tt * c
            rp = c * A[p:p + 1, :] - s * A[q:q + 1, :]
            rq = s * A[p:p + 1, :] + c * A[q:q + 1, :]
            A = jnp.where(ri == p, rp, A)
            A = jnp.where(ri == q, rq, A)
            cp = c * A[:, p:p + 1] - s * A[:, q:q + 1]
            cq = s * A[:, p:p + 1] + c * A[:, q:q + 1]
            A = jnp.where(ci == p, cp, A)
            A = jnp.where(ci == q, cq, A)
            vp = c * V[:, p:p + 1] - s * V[:, q:q + 1]
            vq = s * V[:, p:p + 1] + c * V[:, q:q + 1]
            V = jnp.where(ci == p, vp, V)
            V = jnp.where(ci == q, vq, V)
        return A, V

    A, V = jax.lax.fori_loop(0, NSWEEP, sweep, (A, V0))

    diag = jnp.sum(jnp.where(ri == ci, A, F32(0.0)), axis=0, keepdims=True)
    dmax = jnp.max(diag, axis=1, keepdims=True)
    ci_row = jax.lax.broadcasted_iota(jnp.int32, (1, 4), 1).astype(F32)
    bsel = jnp.min(jnp.where(diag == dmax, ci_row, F32(9.0)),
                   axis=1, keepdims=True)
    onehot = ci_row == bsel
    q = jnp.sum(jnp.where(onehot, V, F32(0.0)), axis=1, keepdims=True)
    q = q / jnp.sqrt(jnp.sum(q * q, axis=0, keepdims=True))

    qw = q[0:1, 0:1]; qx = q[1:2, 0:1]; qy = q[2:3, 0:1]; qz = q[3:4, 0:1]
    r00 = 1.0 - 2.0 * (qy * qy + qz * qz)
    r01 = 2.0 * (qx * qy - qw * qz)
    r02 = 2.0 * (qx * qz + qw * qy)
    r10 = 2.0 * (qx * qy + qw * qz)
    r11 = 1.0 - 2.0 * (qx * qx + qz * qz)
    r12 = 2.0 * (qy * qz - qw * qx)
    r20 = 2.0 * (qx * qz - qw * qy)
    r21 = 2.0 * (qy * qz + qw * qx)
    r22 = 1.0 - 2.0 * (qx * qx + qy * qy)
    t0 = tmx - (r00 * smx + r01 * smy + r02 * smz)
    t1 = tmy - (r10 * smx + r11 * smy + r12 * smz)
    t2 = tmz - (r20 * smx + r21 * smy + r22 * smz)
    R = (r00, r01, r02, r10, r11, r12, r20, r21, r22)
    return R, (t0, t1, t2), q


def _icp_body(src_ref, tgt_ref, tgtp_ref, out_ref, tmp_ref, g_ref):
    tmp_ref[...] = src_ref[0]

    def icp_iter(_, carry):
        def n_body(nt, c2):
            noff = nt * NT
            sx = tmp_ref[0:1, pl.ds(noff, NT)]
            sy = tmp_ref[1:2, pl.ds(noff, NT)]
            sz = tmp_ref[2:3, pl.ds(noff, NT)]
            stile = tmp_ref[:, pl.ds(noff, NT)]
            s2 = sx * sx + sy * sy + sz * sz

            def m_body(c, bd_g):
                bd, G = bd_g
                moff = c * MC
                tx = tgt_ref[0, pl.ds(moff, MC), 0:1]
                ty = tgt_ref[0, pl.ds(moff, MC), 1:2]
                tz = tgt_ref[0, pl.ds(moff, MC), 2:3]
                ttile = tgt_ref[0, pl.ds(moff, MC), :]
                t2 = tx * tx + ty * ty + tz * tz
                cross = jax.lax.dot_general(
                    ttile, stile, (((1,), (0,)), ((), ())),
                    preferred_element_type=F32)
                d2 = jnp.maximum((s2 + t2) - 2.0 * cross, 0.0)
                dmin = jnp.min(d2, axis=0, keepdims=True)
                mi = (jax.lax.broadcasted_iota(jnp.int32, (MC, 1), 0)
                      .astype(F32) + moff.astype(F32))
                idxm = jnp.min(jnp.where(d2 == dmin, mi, F32(BIGF)),
                               axis=0, keepdims=True)
                oh = jnp.where(mi == idxm, F32(1.0), F32(0.0))
                Gc = jax.lax.dot_general(
                    tgtp_ref[0, :, pl.ds(moff, MC)], oh,
                    (((1,), (0,)), ((), ())),
                    precision=jax.lax.Precision.HIGHEST,
                    preferred_element_type=F32)
                upd = dmin < bd
                return (jnp.where(upd, dmin, bd), jnp.where(upd, Gc, G))

            bd0 = jnp.full((1, NT), F32(BIGF))
            g0 = jnp.zeros((3, NT), F32)
            _, G = jax.lax.fori_loop(0, M // MC, m_body, (bd0, g0))
            g_ref[:, pl.ds(noff, NT)] = G
            return c2

        jax.lax.fori_loop(0, N // NT, n_body, 0)

        R, t, _ = _align(
            tmp_ref[0:1, :], tmp_ref[1:2, :], tmp_ref[2:3, :],
            g_ref[0:1, :], g_ref[1:2, :], g_ref[2:3, :])
        (r00, r01, r02, r10, r11, r12, r20, r21, r22) = R
        (t0, t1, t2) = t
        ri3 = jax.lax.broadcasted_iota(jnp.int32, (3, 3), 0)
        ci3 = jax.lax.broadcasted_iota(jnp.int32, (3, 3), 1)
        R33 = jnp.zeros((3, 3), F32)
        for (i, j, v) in ((0, 0, r00), (0, 1, r01), (0, 2, r02),
                          (1, 0, r10), (1, 1, r11), (1, 2, r12),
                          (2, 0, r20), (2, 1, r21), (2, 2, r22)):
            R33 = R33 + jnp.where((ri3 == i) & (ci3 == j), v, F32(0.0))
        newP = jax.lax.dot_general(R33, tmp_ref[...], (((1,), (0,)), ((), ())),
                                   preferred_element_type=F32)
        si = jax.lax.broadcasted_iota(jnp.int32, (3, N), 0)
        tb = jnp.where(si == 0, t0, jnp.where(si == 1, t1, t2))
        tmp_ref[...] = newP + tb
        return carry

    jax.lax.fori_loop(0, ICP_ITERS, icp_iter, 0)

    _, t, q = _align(
        src_ref[0, 0:1, :], src_ref[0, 1:2, :], src_ref[0, 2:3, :],
        tmp_ref[0:1, :], tmp_ref[1:2, :], tmp_ref[2:3, :])
    (t0, t1, t2) = t
    absq = jnp.abs(q)
    qmax = jnp.max(absq, axis=0, keepdims=True)
    ri4 = jax.lax.broadcasted_iota(jnp.int32, (4, 1), 0).astype(F32)
    bsel = jnp.min(jnp.where(absq == qmax, ri4, F32(9.0)),
                   axis=0, keepdims=True)
    val = jnp.sum(jnp.where(ri4 == bsel, q, F32(0.0)), axis=0, keepdims=True)
    q = q * jnp.where(val < 0, F32(-1.0), F32(1.0))
    qw = q[0:1, 0:1]; qx = q[1:2, 0:1]; qy = q[2:3, 0:1]; qz = q[3:4, 0:1]

    li = jax.lax.broadcasted_iota(jnp.int32, (1, 8), 1)
    out = jnp.where(li == 0, t0,
          jnp.where(li == 1, t1,
          jnp.where(li == 2, t2,
          jnp.where(li == 3, qx,
          jnp.where(li == 4, qy,
          jnp.where(li == 5, qz,
          jnp.where(li == 6, qw, F32(0.0))))))))
    out_ref[...] = out.reshape(1, 1, 8)


def kernel(source, target):
    B = source.shape[0]
    src_p = jnp.swapaxes(source, 1, 2)
    tgt_p = jnp.swapaxes(target, 1, 2)
    out3 = pl.pallas_call(
        _icp_body,
        grid=(B,),
        in_specs=[
            pl.BlockSpec((1, 3, N), lambda b: (b, 0, 0)),
            pl.BlockSpec((1, M, 3), lambda b: (b, 0, 0)),
            pl.BlockSpec((1, 3, M), lambda b: (b, 0, 0)),
        ],
        out_specs=pl.BlockSpec((1, 1, 8), lambda b: (b, 0, 0)),
        out_shape=jax.ShapeDtypeStruct((B, 1, 8), F32),
        scratch_shapes=[
            pltpu.VMEM((3, N), F32),
            pltpu.VMEM((3, N), F32),
        ],
        compiler_params=pltpu.CompilerParams(
            dimension_semantics=("parallel",)),
    )(src_p, target, tgt_p)
    return out3[:, 0, :7]
